# baseline (device time: 56652 ns/iter reference)
import jax
import jax.numpy as jnp
from jax import lax
from jax.experimental import pallas as pl
from jax.experimental.pallas import tpu as pltpu

N_DEV = 4


def kernel(x, w_mat, scale_x, scale_w):
    m_total, k_per = x.shape
    _, n = w_mat.shape
    m_per = m_total // N_DEV

    def body(x_ref, w_ref, sx_ref, sw_ref, out_ref, buf_s, buf_r, ssem, rsem):
        p = lax.axis_index("i")
        left = lax.rem(p + (N_DEV - 1), N_DEV)
        right = lax.rem(p + 1, N_DEV)

        barrier_sem = pltpu.get_barrier_semaphore()
        for nbr in (left, right):
            pl.semaphore_signal(
                barrier_sem, inc=1,
                device_id=(nbr,), device_id_type=pl.DeviceIdType.MESH,
            )
        pl.semaphore_wait(barrier_sem, 2)

        rdma = pltpu.make_async_remote_copy(
            src_ref=buf_s, dst_ref=buf_r,
            send_sem=ssem, recv_sem=rsem,
            device_id=(right,), device_id_type=pl.DeviceIdType.MESH,
        )
        rdma.start()

        def partial(c):
            xs = x_ref[pl.ds(c * m_per, m_per), :]
            return lax.dot_general(
                xs, w_ref[:, :],
                dimension_numbers=(((1,), (0,)), ((), ())),
                preferred_element_type=jnp.int32,
            )

        s = sx_ref[0, 0] * sw_ref[0, 0]
        acc = partial(lax.rem(p, N_DEV))
        for k in range(1, N_DEV):
            acc = acc + partial(lax.rem(p + k, N_DEV))
        out_ref[:, :] = acc.astype(jnp.float32) * s

        rdma.wait()

    return pl.pallas_call(
        body,
        out_shape=jax.ShapeDtypeStruct((m_per, n), jnp.float32),
        in_specs=[pl.BlockSpec(memory_space=pltpu.VMEM)] * 4,
        out_specs=pl.BlockSpec(memory_space=pltpu.VMEM),
        scratch_shapes=[
            pltpu.VMEM((2, m_per, 1024), jnp.bfloat16),
            pltpu.VMEM((2, m_per, 1024), jnp.bfloat16),
            pltpu.SemaphoreType.DMA,
            pltpu.SemaphoreType.DMA,
        ],
        compiler_params=pltpu.CompilerParams(collective_id=0),
    )(x, w_mat, scale_x.reshape(1, 1), scale_w.reshape(1, 1))
